# baseline (speedup 1.0000x reference)
import functools

import jax
import jax.numpy as jnp
from jax import lax
from jax.experimental import pallas as pl
from jax.experimental.pallas import tpu as pltpu
from jax.experimental.pallas import tpu_sc as plsc

T = 2048
H = 2048
NH = 16
DH = 128
E = 8
K = 2
DFF = 1024
EPS = 1e-6
P = T * K
BMG = 256
NW = P // BMG
NT = 24

BF16 = jnp.bfloat16
F32 = jnp.float32
I32 = jnp.int32


def _ln_qkv_body(x_ref, g_ref, w_ref, o_ref):
    x = x_ref[...]
    v = jnp.mean(x * x, axis=1, keepdims=True)
    xn = x * jax.lax.rsqrt(v + EPS) * g_ref[...]
    y = jax.lax.dot_general(xn, w_ref[...], (((1,), (0,)), ((), ())),
                            preferred_element_type=F32)
    o_ref[...] = y


def _ln_qkv(x, g, wqkv):
    BN = 512
    return pl.pallas_call(
        _ln_qkv_body,
        grid=(3 * H // BN,),
        in_specs=[
            pl.BlockSpec((T, H), lambda j: (0, 0)),
            pl.BlockSpec((1, H), lambda j: (0, 0)),
            pl.BlockSpec((H, BN), lambda j: (0, j)),
        ],
        out_specs=pl.BlockSpec((T, BN), lambda j: (0, j)),
        out_shape=jax.ShapeDtypeStruct((T, 3 * H), F32),
    )(x, g.reshape(1, H), wqkv)


def _flash_body(q_ref, k_ref, v_ref, o_ref, m_ref, l_ref, acc_ref, *, bq, bk):
    i = pl.program_id(1)
    j = pl.program_id(2)
    scale = 1.0 / (DH ** 0.5)

    @pl.when(j == 0)
    def _():
        m_ref[...] = jnp.full_like(m_ref, -1e30)
        l_ref[...] = jnp.zeros_like(l_ref)
        acc_ref[...] = jnp.zeros_like(acc_ref)

    @pl.when(j <= i)
    def _():
        q = q_ref[...]
        k = k_ref[...]
        s = jax.lax.dot_general(q, k, (((1,), (1,)), ((), ())),
                                preferred_element_type=F32) * scale
        row = i * bq + jax.lax.broadcasted_iota(jnp.int32, (bq, bk), 0)
        col = j * bk + jax.lax.broadcasted_iota(jnp.int32, (bq, bk), 1)
        s = jnp.where(col > row, -1e9, s)
        m_prev = m_ref[:, :1]
        m_cur = jnp.max(s, axis=1, keepdims=True)
        m_new = jnp.maximum(m_prev, m_cur)
        p = jnp.exp(s - m_new)
        corr = jnp.exp(m_prev - m_new)
        l_ref[:, :1] = l_ref[:, :1] * corr + jnp.sum(p, axis=1, keepdims=True)
        acc_ref[...] = acc_ref[...] * corr + jax.lax.dot_general(
            p, v_ref[...], (((1,), (0,)), ((), ())),
            preferred_element_type=F32)
        m_ref[:, :1] = m_new

    @pl.when(j == i)
    def _():
        o_ref[...] = acc_ref[...] / l_ref[:, :1]


def _flash(qkv):
    BQ = 1024
    BK = 1024
    body = functools.partial(_flash_body, bq=BQ, bk=BK)
    return pl.pallas_call(
        body,
        grid=(NH, T // BQ, T // BK),
        in_specs=[
            pl.BlockSpec((BQ, DH), lambda h, i, j: (i, h)),
            pl.BlockSpec((BK, DH), lambda h, i, j: (j, NH + h)),
            pl.BlockSpec((BK, DH), lambda h, i, j: (j, 2 * NH + h)),
        ],
        out_specs=pl.BlockSpec((BQ, DH), lambda h, i, j: (i, h)),
        out_shape=jax.ShapeDtypeStruct((T, H), F32),
        scratch_shapes=[
            pltpu.VMEM((BQ, 1), F32),
            pltpu.VMEM((BQ, 1), F32),
            pltpu.VMEM((BQ, DH), F32),
        ],
        compiler_params=pltpu.CompilerParams(
            dimension_semantics=("arbitrary", "arbitrary", "arbitrary")),
    )(qkv, qkv, qkv)


def _proj_ln2_body(o_ref, x_ref, wo_ref, g_ref, wr_ref,
                   h1_ref, x2n_ref, lg_ref):
    a = jax.lax.dot_general(o_ref[...], wo_ref[...], (((1,), (0,)), ((), ())),
                            preferred_element_type=F32)
    h1 = x_ref[...] + a
    h1_ref[...] = h1
    v = jnp.mean(h1 * h1, axis=1, keepdims=True)
    xn = h1 * jax.lax.rsqrt(v + EPS) * g_ref[...]
    x2n_ref[...] = xn.astype(BF16)
    lg_ref[...] = jax.lax.dot_general(xn, wr_ref[...], (((1,), (0,)), ((), ())),
                                      preferred_element_type=F32)


def _proj_ln2(o, x, wo, g2, wr):
    BM = 256
    return pl.pallas_call(
        _proj_ln2_body,
        grid=(T // BM,),
        in_specs=[
            pl.BlockSpec((BM, H), lambda i: (i, 0)),
            pl.BlockSpec((BM, H), lambda i: (i, 0)),
            pl.BlockSpec((H, H), lambda i: (0, 0)),
            pl.BlockSpec((1, H), lambda i: (0, 0)),
            pl.BlockSpec((H, E), lambda i: (0, 0)),
        ],
        out_specs=[
            pl.BlockSpec((BM, H), lambda i: (i, 0)),
            pl.BlockSpec((BM, H), lambda i: (i, 0)),
            pl.BlockSpec((BM, E), lambda i: (i, 0)),
        ],
        out_shape=[
            jax.ShapeDtypeStruct((T, H), F32),
            jax.ShapeDtypeStruct((T, H), BF16),
            jax.ShapeDtypeStruct((T, E), F32),
        ],
    )(o, x, wo, g2.reshape(1, H), wr)


def _route_plan_body(lg_ref, p0_ref, p1_ref, w0_ref, w1_ref, tiles_ref,
                     c_ref, s1_ref, s2_ref):
    l = lg_ref[...]
    col = jax.lax.broadcasted_iota(I32, (T, E), 1)
    m1 = jnp.max(l, axis=1, keepdims=True)
    a1 = jnp.min(jnp.where(l == m1, col, E), axis=1, keepdims=True)
    sel1 = col == a1
    l2 = jnp.where(sel1, -1e30, l)
    m2 = jnp.max(l2, axis=1, keepdims=True)
    a2 = jnp.min(jnp.where(l2 == m2, col, E), axis=1, keepdims=True)
    sel2 = col == a2
    w0_ref[...] = jax.nn.sigmoid(m1 - m2)
    w1_ref[...] = jax.nn.sigmoid(m2 - m1)

    C = jnp.where(sel1, 1.0, 0.0) + jnp.where(sel2, 1.0, 0.0)
    c_ref[...] = C
    s1_ref[...] = jnp.where(sel1, 1.0, 0.0)
    s2_ref[...] = jnp.where(sel2, 1.0, 0.0)
    tot = jnp.sum(C, axis=0, keepdims=True)
    er = jax.lax.broadcasted_iota(I32, (E, E), 0)
    ec = jax.lax.broadcasted_iota(I32, (E, E), 1)
    u_exc = jnp.where(er < ec, 1.0, 0.0)
    start = jax.lax.dot_general(tot, u_exc, (((1,), (0,)), ((), ())),
                                preferred_element_type=F32,
                                precision=jax.lax.Precision.HIGHEST)

    CH = 256
    rr = jax.lax.broadcasted_iota(I32, (CH, CH), 0)
    cc = jax.lax.broadcasted_iota(I32, (CH, CH), 1)
    l_strict = jnp.where(cc < rr, 1.0, 0.0)

    def chunk(c, carry):
        cC = c_ref[pl.ds(c * CH, CH), :]
        s1 = s1_ref[pl.ds(c * CH, CH), :]
        s2 = s2_ref[pl.ds(c * CH, CH), :]
        rex = jax.lax.dot_general(l_strict, cC, (((1,), (0,)), ((), ())),
                                  preferred_element_type=F32,
                                  precision=jax.lax.Precision.HIGHEST) + carry
        pos = rex + start
        p0_ref[pl.ds(c * CH, CH), :] = jnp.sum(
            pos * s1, axis=1, keepdims=True).astype(I32)
        p1_ref[pl.ds(c * CH, CH), :] = jnp.sum(
            pos * s2, axis=1, keepdims=True).astype(I32)
        return carry + jnp.sum(cC, axis=0, keepdims=True)

    jax.lax.fori_loop(0, T // CH, chunk, jnp.zeros((1, E), F32))

    eye = jnp.where(er == ec, 1.0, 0.0)
    m_exc = jnp.where(ec < er, 1.0, 0.0)
    m_inc = jnp.where(ec <= er, 1.0, 0.0)
    tot_s = jax.lax.dot_general(eye, tot, (((1,), (1,)), ((), ())),
                                preferred_element_type=F32,
                                precision=jax.lax.Precision.HIGHEST)
    lo_s = jax.lax.dot_general(m_exc, tot_s, (((1,), (0,)), ((), ())),
                               preferred_element_type=F32,
                               precision=jax.lax.Precision.HIGHEST)
    tot_i = tot_s.astype(I32)
    lo_i = lo_s.astype(I32)
    hi_i = lo_i + tot_i
    w_lo = lo_i // BMG
    w_hi = (hi_i + (BMG - 1)) // BMG
    n_t = jnp.where(tot_i > 0, w_hi - w_lo, 0)
    n_f = n_t.astype(F32)
    o_exc = jax.lax.dot_general(m_exc, n_f, (((1,), (0,)), ((), ())),
                                preferred_element_type=F32,
                                precision=jax.lax.Precision.HIGHEST).astype(I32)
    cum_in = jax.lax.dot_general(m_inc, n_f, (((1,), (0,)), ((), ())),
                                 preferred_element_type=F32,
                                 precision=jax.lax.Precision.HIGHEST).astype(I32)

    tl = jax.lax.broadcasted_iota(I32, (1, 32), 1)
    e_t = jnp.sum(jnp.where(cum_in <= tl, 1, 0), axis=0, keepdims=True)
    onehot = jax.lax.broadcasted_iota(I32, (E, 32), 0) == e_t
    gat = lambda v: jnp.sum(jnp.where(onehot, v, 0), axis=0, keepdims=True)
    o_t = gat(o_exc)
    wlo_t = gat(w_lo)
    lo_t = gat(lo_i)
    hi_t = gat(hi_i)
    win_t = wlo_t + (tl - o_t)
    dummy = e_t >= E
    win_t = jnp.where(dummy, NW - 1, win_t)
    rlo_t = jnp.where(dummy, P, jnp.maximum(win_t * BMG, lo_t))
    rhi_t = jnp.where(dummy, P, jnp.minimum(win_t * BMG + BMG, hi_t))
    e_t = jnp.where(dummy, E - 1, e_t)
    first_t = jnp.where(rlo_t == win_t * BMG, 1, 0)
    zero = jnp.zeros((3, 32), I32)
    tiles_ref[...] = jnp.concatenate(
        [e_t, win_t, rlo_t, rhi_t, first_t, zero], axis=0)


def _route_plan(logits):
    return pl.pallas_call(
        _route_plan_body,
        out_shape=[
            jax.ShapeDtypeStruct((T, 1), I32),
            jax.ShapeDtypeStruct((T, 1), I32),
            jax.ShapeDtypeStruct((T, 1), F32),
            jax.ShapeDtypeStruct((T, 1), F32),
            jax.ShapeDtypeStruct((8, 32), I32),
        ],
        scratch_shapes=[
            pltpu.VMEM((T, E), F32),
            pltpu.VMEM((T, E), F32),
            pltpu.VMEM((T, E), F32),
        ],
    )(logits)


def _sc_dispatch(x2n_i32, p0, p1):
    mesh = plsc.VectorSubcoreMesh(core_axis_name="c", subcore_axis_name="s")
    tok_w = T // 32

    @functools.partial(
        pl.kernel, mesh=mesh,
        out_type=jax.ShapeDtypeStruct((P, H // 2), I32),
        scratch_types=[
            pltpu.VMEM((tok_w,), I32),
            pltpu.VMEM((tok_w,), I32),
            pltpu.VMEM((tok_w, H // 2), I32),
            pltpu.SemaphoreType.DMA,
        ],
    )
    def k(x_hbm, p0_hbm, p1_hbm, out_hbm, i0_v, i1_v, rows_v, sem):
        wid = lax.axis_index("s") * 2 + lax.axis_index("c")
        base = wid * tok_w
        pltpu.sync_copy(p0_hbm.at[pl.ds(base, tok_w)], i0_v)
        pltpu.sync_copy(p1_hbm.at[pl.ds(base, tok_w)], i1_v)
        pltpu.sync_copy(x_hbm.at[pl.ds(base, tok_w)], rows_v)
        pltpu.async_copy(rows_v, out_hbm.at[i0_v], sem).wait()
        pltpu.async_copy(rows_v, out_hbm.at[i1_v], sem).wait()

    return k(x2n_i32, p0, p1)


def _grouped_body(e_ref, w_ref, lo_ref, hi_ref, fr_ref,
                  xs_ref, w1_ref, w2_ref, out_ref):
    t = pl.program_id(0)
    lo = lo_ref[t]
    hi = hi_ref[t]
    fr = fr_ref[t]
    wn = w_ref[t]
    x = xs_ref[...]
    gu = jax.lax.dot_general(x, w1_ref[0], (((1,), (1,)), ((), ())),
                             preferred_element_type=F32)
    g = gu[:, :DFF]
    u = gu[:, DFF:]
    act = (g * jax.nn.sigmoid(g) * u).astype(BF16)
    dn = jax.lax.dot_general(act, w2_ref[0], (((1,), (1,)), ((), ())),
                             preferred_element_type=F32).astype(BF16)
    rows = wn * BMG + jax.lax.broadcasted_iota(I32, (BMG, 1), 0)
    valid = (rows >= lo) & (rows < hi)

    @pl.when(fr == 1)
    def _():
        out_ref[...] = jnp.where(valid, dn, jnp.zeros_like(dn))

    @pl.when(fr == 0)
    def _():
        out_ref[...] = jnp.where(valid, dn, out_ref[...])


def _grouped(xs, w1b, w2b, e_arr, w_arr, lo_arr, hi_arr, fr_arr):
    grid_spec = pltpu.PrefetchScalarGridSpec(
        num_scalar_prefetch=5,
        grid=(NT,),
        in_specs=[
            pl.BlockSpec((BMG, H), lambda t, e, w, lo, hi, fr: (w[t], 0)),
            pl.BlockSpec((1, 2 * DFF, H),
                         lambda t, e, w, lo, hi, fr: (e[t], 0, 0)),
            pl.BlockSpec((1, H, DFF),
                         lambda t, e, w, lo, hi, fr: (e[t], 0, 0)),
        ],
        out_specs=pl.BlockSpec((BMG, H), lambda t, e, w, lo, hi, fr: (w[t], 0)),
    )
    return pl.pallas_call(
        _grouped_body,
        grid_spec=grid_spec,
        out_shape=jax.ShapeDtypeStruct((P, H), BF16),
        compiler_params=pltpu.CompilerParams(
            dimension_semantics=("arbitrary",)),
    )(e_arr, w_arr, lo_arr, hi_arr, fr_arr, xs, w1b, w2b)


def _sc_combine(ys_i32, p0, p1):
    mesh = plsc.VectorSubcoreMesh(core_axis_name="c", subcore_axis_name="s")
    tok_w = T // 32
    CH = 32

    @functools.partial(
        pl.kernel, mesh=mesh,
        out_type=[
            jax.ShapeDtypeStruct((T, H // 2), I32),
            jax.ShapeDtypeStruct((T, H // 2), I32),
        ],
        scratch_types=[
            pltpu.VMEM((CH,), I32),
            pltpu.VMEM((CH,), I32),
            pltpu.VMEM((CH, H // 2), I32),
            pltpu.VMEM((CH, H // 2), I32),
            pltpu.SemaphoreType.DMA,
        ],
    )
    def k(ys_hbm, p0_hbm, p1_hbm, g0_hbm, g1_hbm,
          i0_v, i1_v, b0_v, b1_v, sem):
        wid = lax.axis_index("s") * 2 + lax.axis_index("c")
        base = wid * tok_w

        def chunk(c, _):
            off = base + c * CH
            pltpu.sync_copy(p0_hbm.at[pl.ds(off, CH)], i0_v)
            pltpu.sync_copy(p1_hbm.at[pl.ds(off, CH)], i1_v)
            pltpu.async_copy(ys_hbm.at[i0_v], b0_v, sem).wait()
            pltpu.async_copy(ys_hbm.at[i1_v], b1_v, sem).wait()
            pltpu.sync_copy(b0_v, g0_hbm.at[pl.ds(off, CH)])
            pltpu.sync_copy(b1_v, g1_hbm.at[pl.ds(off, CH)])
            return 0

        lax.fori_loop(0, tok_w // CH, chunk, 0)

    return k(ys_i32, p0, p1)


def _final_body(h1_ref, g0_ref, g1_ref, w0_ref, w1_ref, o_ref):
    o_ref[...] = (h1_ref[...]
                  + w0_ref[...] * g0_ref[...].astype(F32)
                  + w1_ref[...] * g1_ref[...].astype(F32))


def _final(h1, g0, g1, w0, w1):
    BM = 512
    return pl.pallas_call(
        _final_body,
        grid=(T // BM,),
        in_specs=[
            pl.BlockSpec((BM, H), lambda i: (i, 0)),
            pl.BlockSpec((BM, H), lambda i: (i, 0)),
            pl.BlockSpec((BM, H), lambda i: (i, 0)),
            pl.BlockSpec((BM, 1), lambda i: (i, 0)),
            pl.BlockSpec((BM, 1), lambda i: (i, 0)),
        ],
        out_specs=pl.BlockSpec((BM, H), lambda i: (i, 0)),
        out_shape=jax.ShapeDtypeStruct((T, H), F32),
    )(h1, g0, g1, w0, w1)


def _bf16_to_i32(x):
    return jax.lax.bitcast_convert_type(
        x.reshape(x.shape[0], x.shape[1] // 2, 2), I32)


def _i32_to_bf16(x):
    return jax.lax.bitcast_convert_type(x, BF16).reshape(
        x.shape[0], x.shape[1] * 2)


def kernel(hidden_states, ln1_g, ln2_g, wq, wk, wv, wo, w_router, w1, w2):
    wqkv = jnp.concatenate([wq, wk, wv], axis=1)
    qkv = _ln_qkv(hidden_states, ln1_g, wqkv)
    o = _flash(qkv)
    h1, x2n, logits = _proj_ln2(o, hidden_states, wo, ln2_g, w_router)
    p0c, p1c, w0c, w1c, tiles = _route_plan(logits)
    p0 = p0c.reshape(T)
    p1 = p1c.reshape(T)
    xs_i32 = _sc_dispatch(_bf16_to_i32(x2n), p0, p1)
    xs = _i32_to_bf16(xs_i32)
    ys = _grouped(xs, w1.astype(BF16), w2.astype(BF16),
                  tiles[0, :NT], tiles[1, :NT], tiles[2, :NT],
                  tiles[3, :NT], tiles[4, :NT])
    g0_i32, g1_i32 = _sc_combine(_bf16_to_i32(ys), p0, p1)
    out = _final(h1, _i32_to_bf16(g0_i32), _i32_to_bf16(g1_i32), w0c, w1c)
    return out


# speedup vs baseline: 1.3482x; 1.3482x over previous
import functools

import jax
import jax.numpy as jnp
from jax import lax
from jax.experimental import pallas as pl
from jax.experimental.pallas import tpu as pltpu
from jax.experimental.pallas import tpu_sc as plsc

T = 2048
H = 2048
NH = 16
DH = 128
E = 8
K = 2
DFF = 1024
EPS = 1e-6
P = T * K
BMG = 256
NW = P // BMG
NT = 24

BF16 = jnp.bfloat16
F32 = jnp.float32
I32 = jnp.int32


def _ln_qkv_body(x_ref, g_ref, w_ref, o_ref):
    x = x_ref[...]
    v = jnp.mean(x * x, axis=1, keepdims=True)
    xn = x * jax.lax.rsqrt(v + EPS) * g_ref[...]
    y = jax.lax.dot_general(xn, w_ref[...], (((1,), (0,)), ((), ())),
                            preferred_element_type=F32)
    o_ref[...] = y


def _ln_qkv(x, g, wqkv):
    BN = 512
    return pl.pallas_call(
        _ln_qkv_body,
        grid=(3 * H // BN,),
        in_specs=[
            pl.BlockSpec((T, H), lambda j: (0, 0)),
            pl.BlockSpec((1, H), lambda j: (0, 0)),
            pl.BlockSpec((H, BN), lambda j: (0, j)),
        ],
        out_specs=pl.BlockSpec((T, BN), lambda j: (0, j)),
        out_shape=jax.ShapeDtypeStruct((T, 3 * H), F32),
    )(x, g.reshape(1, H), wqkv)


def _flash_body(q_ref, k_ref, v_ref, o_ref, m_ref, l_ref, acc_ref, *, bq, bk):
    i = pl.program_id(1)
    j = pl.program_id(2)
    scale = 1.0 / (DH ** 0.5)

    @pl.when(j == 0)
    def _():
        m_ref[...] = jnp.full_like(m_ref, -1e30)
        l_ref[...] = jnp.zeros_like(l_ref)
        acc_ref[...] = jnp.zeros_like(acc_ref)

    @pl.when(j <= i)
    def _():
        q = q_ref[...]
        k = k_ref[...]
        s = jax.lax.dot_general(q, k, (((1,), (1,)), ((), ())),
                                preferred_element_type=F32) * scale
        row = i * bq + jax.lax.broadcasted_iota(jnp.int32, (bq, bk), 0)
        col = j * bk + jax.lax.broadcasted_iota(jnp.int32, (bq, bk), 1)
        s = jnp.where(col > row, -1e9, s)
        m_prev = m_ref[:, :1]
        m_cur = jnp.max(s, axis=1, keepdims=True)
        m_new = jnp.maximum(m_prev, m_cur)
        p = jnp.exp(s - m_new)
        corr = jnp.exp(m_prev - m_new)
        l_ref[:, :1] = l_ref[:, :1] * corr + jnp.sum(p, axis=1, keepdims=True)
        acc_ref[...] = acc_ref[...] * corr + jax.lax.dot_general(
            p, v_ref[...], (((1,), (0,)), ((), ())),
            preferred_element_type=F32)
        m_ref[:, :1] = m_new

    @pl.when(j == i)
    def _():
        o_ref[...] = acc_ref[...] / l_ref[:, :1]


def _flash(qkv):
    BQ = 1024
    BK = 1024
    body = functools.partial(_flash_body, bq=BQ, bk=BK)
    return pl.pallas_call(
        body,
        grid=(NH, T // BQ, T // BK),
        in_specs=[
            pl.BlockSpec((BQ, DH), lambda h, i, j: (i, h)),
            pl.BlockSpec((BK, DH), lambda h, i, j: (j, NH + h)),
            pl.BlockSpec((BK, DH), lambda h, i, j: (j, 2 * NH + h)),
        ],
        out_specs=pl.BlockSpec((BQ, DH), lambda h, i, j: (i, h)),
        out_shape=jax.ShapeDtypeStruct((T, H), F32),
        scratch_shapes=[
            pltpu.VMEM((BQ, 1), F32),
            pltpu.VMEM((BQ, 1), F32),
            pltpu.VMEM((BQ, DH), F32),
        ],
        compiler_params=pltpu.CompilerParams(
            dimension_semantics=("arbitrary", "arbitrary", "arbitrary")),
    )(qkv, qkv, qkv)


def _proj_ln2_body(o_ref, x_ref, wo_ref, g_ref, wr_ref,
                   h1_ref, x2n_ref, lg_ref):
    a = jax.lax.dot_general(o_ref[...], wo_ref[...], (((1,), (0,)), ((), ())),
                            preferred_element_type=F32)
    h1 = x_ref[...] + a
    h1_ref[...] = h1
    v = jnp.mean(h1 * h1, axis=1, keepdims=True)
    xn = h1 * jax.lax.rsqrt(v + EPS) * g_ref[...]
    x2n_ref[...] = xn.astype(BF16)
    lg_ref[...] = jax.lax.dot_general(xn, wr_ref[...], (((1,), (0,)), ((), ())),
                                      preferred_element_type=F32)


def _proj_ln2(o, x, wo, g2, wr):
    BM = 256
    return pl.pallas_call(
        _proj_ln2_body,
        grid=(T // BM,),
        in_specs=[
            pl.BlockSpec((BM, H), lambda i: (i, 0)),
            pl.BlockSpec((BM, H), lambda i: (i, 0)),
            pl.BlockSpec((H, H), lambda i: (0, 0)),
            pl.BlockSpec((1, H), lambda i: (0, 0)),
            pl.BlockSpec((H, E), lambda i: (0, 0)),
        ],
        out_specs=[
            pl.BlockSpec((BM, H), lambda i: (i, 0)),
            pl.BlockSpec((BM, H), lambda i: (i, 0)),
            pl.BlockSpec((BM, E), lambda i: (i, 0)),
        ],
        out_shape=[
            jax.ShapeDtypeStruct((T, H), F32),
            jax.ShapeDtypeStruct((T, H), BF16),
            jax.ShapeDtypeStruct((T, E), F32),
        ],
    )(o, x, wo, g2.reshape(1, H), wr)


def _route_plan_body(lg_ref, p0_ref, p1_ref, w0_ref, w1_ref, tiles_ref,
                     c_ref, s1_ref, s2_ref):
    l = lg_ref[...]
    col = jax.lax.broadcasted_iota(I32, (T, E), 1)
    m1 = jnp.max(l, axis=1, keepdims=True)
    a1 = jnp.min(jnp.where(l == m1, col, E), axis=1, keepdims=True)
    sel1 = col == a1
    l2 = jnp.where(sel1, -1e30, l)
    m2 = jnp.max(l2, axis=1, keepdims=True)
    a2 = jnp.min(jnp.where(l2 == m2, col, E), axis=1, keepdims=True)
    sel2 = col == a2
    w0_ref[...] = jax.nn.sigmoid(m1 - m2)
    w1_ref[...] = jax.nn.sigmoid(m2 - m1)

    C = jnp.where(sel1, 1.0, 0.0) + jnp.where(sel2, 1.0, 0.0)
    c_ref[...] = C
    s1_ref[...] = jnp.where(sel1, 1.0, 0.0)
    s2_ref[...] = jnp.where(sel2, 1.0, 0.0)
    tot = jnp.sum(C, axis=0, keepdims=True)
    er = jax.lax.broadcasted_iota(I32, (E, E), 0)
    ec = jax.lax.broadcasted_iota(I32, (E, E), 1)
    u_exc = jnp.where(er < ec, 1.0, 0.0)
    start = jax.lax.dot_general(tot, u_exc, (((1,), (0,)), ((), ())),
                                preferred_element_type=F32,
                                precision=jax.lax.Precision.HIGHEST)

    CH = 256
    rr = jax.lax.broadcasted_iota(I32, (CH, CH), 0)
    cc = jax.lax.broadcasted_iota(I32, (CH, CH), 1)
    l_strict = jnp.where(cc < rr, 1.0, 0.0)

    def chunk(c, carry):
        cC = c_ref[pl.ds(c * CH, CH), :]
        s1 = s1_ref[pl.ds(c * CH, CH), :]
        s2 = s2_ref[pl.ds(c * CH, CH), :]
        rex = jax.lax.dot_general(l_strict, cC, (((1,), (0,)), ((), ())),
                                  preferred_element_type=F32,
                                  precision=jax.lax.Precision.HIGHEST) + carry
        pos = rex + start
        p0_ref[pl.ds(c * CH, CH), :] = jnp.sum(
            pos * s1, axis=1, keepdims=True).astype(I32)
        p1_ref[pl.ds(c * CH, CH), :] = jnp.sum(
            pos * s2, axis=1, keepdims=True).astype(I32)
        return carry + jnp.sum(cC, axis=0, keepdims=True)

    jax.lax.fori_loop(0, T // CH, chunk, jnp.zeros((1, E), F32))

    eye = jnp.where(er == ec, 1.0, 0.0)
    m_exc = jnp.where(ec < er, 1.0, 0.0)
    m_inc = jnp.where(ec <= er, 1.0, 0.0)
    tot_s = jax.lax.dot_general(eye, tot, (((1,), (1,)), ((), ())),
                                preferred_element_type=F32,
                                precision=jax.lax.Precision.HIGHEST)
    lo_s = jax.lax.dot_general(m_exc, tot_s, (((1,), (0,)), ((), ())),
                               preferred_element_type=F32,
                               precision=jax.lax.Precision.HIGHEST)
    tot_i = tot_s.astype(I32)
    lo_i = lo_s.astype(I32)
    hi_i = lo_i + tot_i
    w_lo = lo_i // BMG
    w_hi = (hi_i + (BMG - 1)) // BMG
    n_t = jnp.where(tot_i > 0, w_hi - w_lo, 0)
    n_f = n_t.astype(F32)
    o_exc = jax.lax.dot_general(m_exc, n_f, (((1,), (0,)), ((), ())),
                                preferred_element_type=F32,
                                precision=jax.lax.Precision.HIGHEST).astype(I32)
    cum_in = jax.lax.dot_general(m_inc, n_f, (((1,), (0,)), ((), ())),
                                 preferred_element_type=F32,
                                 precision=jax.lax.Precision.HIGHEST).astype(I32)

    tl = jax.lax.broadcasted_iota(I32, (1, 32), 1)
    e_t = jnp.sum(jnp.where(cum_in <= tl, 1, 0), axis=0, keepdims=True)
    onehot = jax.lax.broadcasted_iota(I32, (E, 32), 0) == e_t
    gat = lambda v: jnp.sum(jnp.where(onehot, v, 0), axis=0, keepdims=True)
    o_t = gat(o_exc)
    wlo_t = gat(w_lo)
    lo_t = gat(lo_i)
    hi_t = gat(hi_i)
    win_t = wlo_t + (tl - o_t)
    dummy = e_t >= E
    win_t = jnp.where(dummy, NW - 1, win_t)
    rlo_t = jnp.where(dummy, P, jnp.maximum(win_t * BMG, lo_t))
    rhi_t = jnp.where(dummy, P, jnp.minimum(win_t * BMG + BMG, hi_t))
    e_t = jnp.where(dummy, E - 1, e_t)
    first_t = jnp.where(rlo_t == win_t * BMG, 1, 0)
    zero = jnp.zeros((3, 32), I32)
    tiles_ref[...] = jnp.concatenate(
        [e_t, win_t, rlo_t, rhi_t, first_t, zero], axis=0)


def _route_plan(logits):
    return pl.pallas_call(
        _route_plan_body,
        out_shape=[
            jax.ShapeDtypeStruct((T, 1), I32),
            jax.ShapeDtypeStruct((T, 1), I32),
            jax.ShapeDtypeStruct((T, 1), F32),
            jax.ShapeDtypeStruct((T, 1), F32),
            jax.ShapeDtypeStruct((8, 32), I32),
        ],
        scratch_shapes=[
            pltpu.VMEM((T, E), F32),
            pltpu.VMEM((T, E), F32),
            pltpu.VMEM((T, E), F32),
        ],
    )(logits)


def _sc_dispatch(x2n_i32, p0, p1):
    mesh = plsc.VectorSubcoreMesh(core_axis_name="c", subcore_axis_name="s")
    tok_w = T // 32

    @functools.partial(
        pl.kernel, mesh=mesh,
        out_type=jax.ShapeDtypeStruct((P, H // 2), I32),
        scratch_types=[
            pltpu.VMEM((tok_w,), I32),
            pltpu.VMEM((tok_w,), I32),
            pltpu.VMEM((tok_w, H // 2), I32),
            pltpu.SemaphoreType.DMA,
        ],
    )
    def k(x_hbm, p0_hbm, p1_hbm, out_hbm, i0_v, i1_v, rows_v, sem):
        wid = lax.axis_index("s") * 2 + lax.axis_index("c")
        base = wid * tok_w
        pltpu.sync_copy(p0_hbm.at[pl.ds(base, tok_w)], i0_v)
        pltpu.sync_copy(p1_hbm.at[pl.ds(base, tok_w)], i1_v)
        pltpu.sync_copy(x_hbm.at[pl.ds(base, tok_w)], rows_v)
        pltpu.async_copy(rows_v, out_hbm.at[i0_v], sem).wait()
        pltpu.async_copy(rows_v, out_hbm.at[i1_v], sem).wait()

    return k(x2n_i32, p0, p1)


def _grouped_body(e_ref, w_ref, lo_ref, hi_ref, fr_ref,
                  xs_ref, w1_ref, w2_ref, out_ref):
    t = pl.program_id(0)
    lo = lo_ref[t]
    hi = hi_ref[t]
    fr = fr_ref[t]
    wn = w_ref[t]
    x = xs_ref[...]
    gu = jax.lax.dot_general(x, w1_ref[0], (((1,), (1,)), ((), ())),
                             preferred_element_type=F32)
    g = gu[:, :DFF]
    u = gu[:, DFF:]
    act = (g * jax.nn.sigmoid(g) * u).astype(BF16)
    dn = jax.lax.dot_general(act, w2_ref[0], (((1,), (1,)), ((), ())),
                             preferred_element_type=F32).astype(BF16)
    rows = wn * BMG + jax.lax.broadcasted_iota(I32, (BMG, 1), 0)
    valid = (rows >= lo) & (rows < hi)

    @pl.when(fr == 1)
    def _():
        out_ref[...] = jnp.where(valid, dn, jnp.zeros_like(dn))

    @pl.when(fr == 0)
    def _():
        out_ref[...] = jnp.where(valid, dn, out_ref[...])


def _grouped(xs, w1b, w2b, e_arr, w_arr, lo_arr, hi_arr, fr_arr):
    grid_spec = pltpu.PrefetchScalarGridSpec(
        num_scalar_prefetch=5,
        grid=(NT,),
        in_specs=[
            pl.BlockSpec((BMG, H), lambda t, e, w, lo, hi, fr: (w[t], 0)),
            pl.BlockSpec((1, 2 * DFF, H),
                         lambda t, e, w, lo, hi, fr: (e[t], 0, 0)),
            pl.BlockSpec((1, H, DFF),
                         lambda t, e, w, lo, hi, fr: (e[t], 0, 0)),
        ],
        out_specs=pl.BlockSpec((BMG, H), lambda t, e, w, lo, hi, fr: (w[t], 0)),
    )
    return pl.pallas_call(
        _grouped_body,
        grid_spec=grid_spec,
        out_shape=jax.ShapeDtypeStruct((P, H), BF16),
        compiler_params=pltpu.CompilerParams(
            dimension_semantics=("arbitrary",)),
    )(e_arr, w_arr, lo_arr, hi_arr, fr_arr, xs, w1b, w2b)


def _combine_body(h1_ref, ys_ref, p0_ref, p1_ref, w0_ref, w1_ref, o_ref,
                  *, bm):
    s_iota = jax.lax.broadcasted_iota(I32, (bm, P), 1)
    g = jnp.where(s_iota == p0_ref[...], w0_ref[...], 0.0)
    g = g + jnp.where(s_iota == p1_ref[...], w1_ref[...], 0.0)
    acc = jax.lax.dot_general(g.astype(BF16), ys_ref[...],
                              (((1,), (0,)), ((), ())),
                              preferred_element_type=F32)
    o_ref[...] = h1_ref[...] + acc


def _combine(h1, ys, p0c, p1c, w0c, w1c):
    BM = 256
    body = functools.partial(_combine_body, bm=BM)
    return pl.pallas_call(
        body,
        grid=(T // BM,),
        in_specs=[
            pl.BlockSpec((BM, H), lambda i: (i, 0)),
            pl.BlockSpec((P, H), lambda i: (0, 0)),
            pl.BlockSpec((BM, 1), lambda i: (i, 0)),
            pl.BlockSpec((BM, 1), lambda i: (i, 0)),
            pl.BlockSpec((BM, 1), lambda i: (i, 0)),
            pl.BlockSpec((BM, 1), lambda i: (i, 0)),
        ],
        out_specs=pl.BlockSpec((BM, H), lambda i: (i, 0)),
        out_shape=jax.ShapeDtypeStruct((T, H), F32),
    )(h1, ys, p0c, p1c, w0c, w1c)


def _bf16_to_i32(x):
    return jax.lax.bitcast_convert_type(
        x.reshape(x.shape[0], x.shape[1] // 2, 2), I32)


def _i32_to_bf16(x):
    return jax.lax.bitcast_convert_type(x, BF16).reshape(
        x.shape[0], x.shape[1] * 2)


def kernel(hidden_states, ln1_g, ln2_g, wq, wk, wv, wo, w_router, w1, w2):
    wqkv = jnp.concatenate([wq, wk, wv], axis=1)
    qkv = _ln_qkv(hidden_states, ln1_g, wqkv)
    o = _flash(qkv)
    h1, x2n, logits = _proj_ln2(o, hidden_states, wo, ln2_g, w_router)
    p0c, p1c, w0c, w1c, tiles = _route_plan(logits)
    p0 = p0c.reshape(T)
    p1 = p1c.reshape(T)
    xs_i32 = _sc_dispatch(_bf16_to_i32(x2n), p0, p1)
    xs = _i32_to_bf16(xs_i32)
    ys = _grouped(xs, w1.astype(BF16), w2.astype(BF16),
                  tiles[0, :NT], tiles[1, :NT], tiles[2, :NT],
                  tiles[3, :NT], tiles[4, :NT])
    out = _combine(h1, ys, p0c, p1c, w0c, w1c)
    return out


# speedup vs baseline: 1.6834x; 1.2486x over previous
import functools

import jax
import jax.numpy as jnp
from jax import lax
from jax.experimental import pallas as pl
from jax.experimental.pallas import tpu as pltpu
from jax.experimental.pallas import tpu_sc as plsc

T = 2048
H = 2048
NH = 16
DH = 128
E = 8
K = 2
DFF = 1024
EPS = 1e-6
P = T * K
BMG = 256
NW = P // BMG
NT = 24

BF16 = jnp.bfloat16
F32 = jnp.float32
I32 = jnp.int32


def _ln_qkv_body(x_ref, g_ref, w_ref, o_ref):
    x = x_ref[...]
    v = jnp.mean(x * x, axis=1, keepdims=True)
    xn = x * jax.lax.rsqrt(v + EPS) * g_ref[...]
    y = jax.lax.dot_general(xn, w_ref[...], (((1,), (0,)), ((), ())),
                            preferred_element_type=F32)
    o_ref[...] = y


def _ln_qkv(x, g, wqkv):
    BN = 512
    return pl.pallas_call(
        _ln_qkv_body,
        grid=(3 * H // BN,),
        in_specs=[
            pl.BlockSpec((T, H), lambda j: (0, 0)),
            pl.BlockSpec((1, H), lambda j: (0, 0)),
            pl.BlockSpec((H, BN), lambda j: (0, j)),
        ],
        out_specs=pl.BlockSpec((T, BN), lambda j: (0, j)),
        out_shape=jax.ShapeDtypeStruct((T, 3 * H), F32),
    )(x, g.reshape(1, H), wqkv)


def _flash_body(q_ref, k_ref, v_ref, o_ref, m_ref, l_ref, acc_ref, *, bq, bk):
    i = pl.program_id(1)
    j = pl.program_id(2)
    scale = 1.0 / (DH ** 0.5)

    @pl.when(j == 0)
    def _():
        m_ref[...] = jnp.full_like(m_ref, -1e30)
        l_ref[...] = jnp.zeros_like(l_ref)
        acc_ref[...] = jnp.zeros_like(acc_ref)

    @pl.when(j <= i)
    def _():
        q = q_ref[...]
        k = k_ref[...]
        s = jax.lax.dot_general(q, k, (((1,), (1,)), ((), ())),
                                preferred_element_type=F32) * scale
        row = i * bq + jax.lax.broadcasted_iota(jnp.int32, (bq, bk), 0)
        col = j * bk + jax.lax.broadcasted_iota(jnp.int32, (bq, bk), 1)
        s = jnp.where(col > row, -1e9, s)
        m_prev = m_ref[:, :1]
        m_cur = jnp.max(s, axis=1, keepdims=True)
        m_new = jnp.maximum(m_prev, m_cur)
        p = jnp.exp(s - m_new)
        corr = jnp.exp(m_prev - m_new)
        l_ref[:, :1] = l_ref[:, :1] * corr + jnp.sum(p, axis=1, keepdims=True)
        acc_ref[...] = acc_ref[...] * corr + jax.lax.dot_general(
            p, v_ref[...], (((1,), (0,)), ((), ())),
            preferred_element_type=F32)
        m_ref[:, :1] = m_new

    @pl.when(j == i)
    def _():
        o_ref[...] = acc_ref[...] / l_ref[:, :1]


def _flash(qkv):
    BQ = 1024
    BK = 1024
    body = functools.partial(_flash_body, bq=BQ, bk=BK)
    return pl.pallas_call(
        body,
        grid=(NH, T // BQ, T // BK),
        in_specs=[
            pl.BlockSpec((BQ, DH), lambda h, i, j: (i, h)),
            pl.BlockSpec((BK, DH), lambda h, i, j: (j, NH + h)),
            pl.BlockSpec((BK, DH), lambda h, i, j: (j, 2 * NH + h)),
        ],
        out_specs=pl.BlockSpec((BQ, DH), lambda h, i, j: (i, h)),
        out_shape=jax.ShapeDtypeStruct((T, H), F32),
        scratch_shapes=[
            pltpu.VMEM((BQ, 1), F32),
            pltpu.VMEM((BQ, 1), F32),
            pltpu.VMEM((BQ, DH), F32),
        ],
        compiler_params=pltpu.CompilerParams(
            dimension_semantics=("arbitrary", "arbitrary", "arbitrary")),
    )(qkv, qkv, qkv)


def _proj_ln2_body(o_ref, x_ref, wo_ref, g_ref, wr_ref,
                   h1_ref, x2n_ref, lg_ref):
    a = jax.lax.dot_general(o_ref[...], wo_ref[...], (((1,), (0,)), ((), ())),
                            preferred_element_type=F32)
    h1 = x_ref[...] + a
    h1_ref[...] = h1
    v = jnp.mean(h1 * h1, axis=1, keepdims=True)
    xn = h1 * jax.lax.rsqrt(v + EPS) * g_ref[...]
    x2n_ref[...] = xn.astype(BF16)
    lg_ref[...] = jax.lax.dot_general(xn, wr_ref[...], (((1,), (0,)), ((), ())),
                                      preferred_element_type=F32)


def _proj_ln2(o, x, wo, g2, wr):
    BM = 256
    return pl.pallas_call(
        _proj_ln2_body,
        grid=(T // BM,),
        in_specs=[
            pl.BlockSpec((BM, H), lambda i: (i, 0)),
            pl.BlockSpec((BM, H), lambda i: (i, 0)),
            pl.BlockSpec((H, H), lambda i: (0, 0)),
            pl.BlockSpec((1, H), lambda i: (0, 0)),
            pl.BlockSpec((H, E), lambda i: (0, 0)),
        ],
        out_specs=[
            pl.BlockSpec((BM, H), lambda i: (i, 0)),
            pl.BlockSpec((BM, H), lambda i: (i, 0)),
            pl.BlockSpec((BM, E), lambda i: (i, 0)),
        ],
        out_shape=[
            jax.ShapeDtypeStruct((T, H), F32),
            jax.ShapeDtypeStruct((T, H), BF16),
            jax.ShapeDtypeStruct((T, E), F32),
        ],
    )(o, x, wo, g2.reshape(1, H), wr)


def _route_body(lg_ref, comb_ref):
    l = lg_ref[...]
    col = jax.lax.broadcasted_iota(I32, (T, E), 1)
    m1 = jnp.max(l, axis=1, keepdims=True)
    a1 = jnp.min(jnp.where(l == m1, col, E), axis=1, keepdims=True)
    sel1 = col == a1
    l2 = jnp.where(sel1, -1e30, l)
    m2 = jnp.max(l2, axis=1, keepdims=True)
    a2 = jnp.min(jnp.where(l2 == m2, col, E), axis=1, keepdims=True)
    sel2 = col == a2
    w1 = jax.nn.sigmoid(m1 - m2)
    w2 = 1.0 - w1
    comb_ref[...] = jnp.where(sel1, w1, 0.0) + jnp.where(sel2, w2, 0.0)


def _route(logits):
    return pl.pallas_call(
        _route_body,
        out_shape=jax.ShapeDtypeStruct((T, E), F32),
    )(logits)


def _moe_body(x_ref, w1_ref, w2_ref, comb_ref, h1_ref, o_ref):
    e = pl.program_id(1)
    x = x_ref[...]
    gu = jax.lax.dot_general(x, w1_ref[0], (((1,), (1,)), ((), ())),
                             preferred_element_type=F32)
    g = gu[:, :DFF]
    u = gu[:, DFF:]
    act = (g * jax.nn.sigmoid(g) * u).astype(BF16)
    dn = jax.lax.dot_general(act, w2_ref[0], (((1,), (1,)), ((), ())),
                             preferred_element_type=F32)
    c = comb_ref[...]
    onehot = (jax.lax.broadcasted_iota(I32, c.shape, 1) == e)
    wgt = jnp.sum(jnp.where(onehot, c, 0.0), axis=1, keepdims=True)
    contrib = dn * wgt

    @pl.when(e == 0)
    def _():
        o_ref[...] = h1_ref[...] + contrib

    @pl.when(e > 0)
    def _():
        o_ref[...] = o_ref[...] + contrib


def _moe(x2n, w1, w2, comb, h1):
    BM = 512
    return pl.pallas_call(
        _moe_body,
        grid=(T // BM, E),
        in_specs=[
            pl.BlockSpec((BM, H), lambda i, e: (i, 0)),
            pl.BlockSpec((1, 2 * DFF, H), lambda i, e: (e, 0, 0)),
            pl.BlockSpec((1, H, DFF), lambda i, e: (e, 0, 0)),
            pl.BlockSpec((BM, E), lambda i, e: (i, 0)),
            pl.BlockSpec((BM, H), lambda i, e: (i, 0)),
        ],
        out_specs=pl.BlockSpec((BM, H), lambda i, e: (i, 0)),
        out_shape=jax.ShapeDtypeStruct((T, H), F32),
        compiler_params=pltpu.CompilerParams(
            dimension_semantics=("arbitrary", "arbitrary")),
    )(x2n, w1, w2, comb, h1)


def kernel(hidden_states, ln1_g, ln2_g, wq, wk, wv, wo, w_router, w1, w2):
    wqkv = jnp.concatenate([wq, wk, wv], axis=1)
    qkv = _ln_qkv(hidden_states, ln1_g, wqkv)
    o = _flash(qkv)
    h1, x2n, logits = _proj_ln2(o, hidden_states, wo, ln2_g, w_router)
    comb = _route(logits)
    out = _moe(x2n, w1.astype(BF16), w2.astype(BF16), comb, h1)
    return out
